# Initial kernel scaffold; baseline (speedup 1.0000x reference)
import jax
import jax.numpy as jnp
from jax.experimental import pallas as pl


def kernel(x, coefficients_vect, scaling_coeffs_vect, zero_knot_indexes):
    raise NotImplementedError("write your pallas kernel here")


# speedup vs baseline: 613.8510x; 613.8510x over previous
import jax
import jax.numpy as jnp
from jax import lax
from jax.experimental import pallas as pl
from jax.experimental.pallas import tpu as pltpu
from jax.experimental.pallas import tpu_sc as plsc

_NUM_ACT = 96
_SIZE = 101
_HALF = _SIZE // 2
_INV_GRID = 12.5

_N = 4 * _NUM_ACT * 224 * 224
_NTILES = 32
_PER_TILE = _N // _NTILES
_SLAB = 224 * 224
_CH = _SLAB // 2
_CHUNKS = _PER_TILE // _CH
_VREGS = _CH // 16


def _tec_body(x_hbm, cv_hbm, sc_hbm, zk_hbm, out_hbm,
              xbuf, obuf, cv_v, sc_v, zk_v):
    info = plsc.get_sparse_core_info()
    nc = info.num_cores
    wid = lax.axis_index("s") * nc + lax.axis_index("c")

    pltpu.sync_copy(cv_hbm, cv_v.at[pl.ds(0, _NUM_ACT * _SIZE)])
    pltpu.sync_copy(sc_hbm, sc_v.at[pl.ds(0, _NUM_ACT)])
    pltpu.sync_copy(zk_hbm, zk_v.at[pl.ds(0, _NUM_ACT)])

    def chunk(k, carry):
        off = wid * _PER_TILE + k * _CH
        pltpu.sync_copy(x_hbm.at[pl.ds(off, _CH)], xbuf)

        c = ((wid * _CHUNKS + k) // (_SLAB // _CH)) % _NUM_ACT
        ci = jnp.full((16,), c, jnp.int32)
        scv = plsc.load_gather(sc_v, [ci])
        zkv = plsc.load_gather(zk_v, [ci])
        a = scv * _INV_GRID
        rs = 1.0 / scv
        base = zkv - _HALF

        def vec(i, carry2):
            xv = xbuf[pl.ds(i * 16, 16)]
            s = xv * a + float(_HALF)
            u = jnp.clip(s, 0.0, float(_SIZE - 2))
            iu = u.astype(jnp.int32)
            fu = iu.astype(jnp.float32)
            fr = s - fu
            ix = base + iu
            g0 = plsc.load_gather(cv_v, [ix])
            g1 = plsc.load_gather(cv_v, [ix + 1])
            obuf[pl.ds(i * 16, 16)] = (g1 * fr + g0 * (1.0 - fr)) * rs
            return carry2

        lax.fori_loop(0, _VREGS, vec, 0)
        pltpu.sync_copy(obuf, out_hbm.at[pl.ds(off, _CH)])
        return carry

    lax.fori_loop(0, _CHUNKS, chunk, 0)


@jax.jit
def _spline_sc(xflat, cv, scv, zk):
    run = pl.kernel(
        _tec_body,
        out_type=jax.ShapeDtypeStruct((_N,), jnp.float32),
        mesh=plsc.VectorSubcoreMesh(core_axis_name="c", subcore_axis_name="s"),
        compiler_params=pltpu.CompilerParams(needs_layout_passes=False),
        scratch_types=[
            pltpu.VMEM((_CH,), jnp.float32),
            pltpu.VMEM((_CH,), jnp.float32),
            pltpu.VMEM((9728,), jnp.float32),
            pltpu.VMEM((128,), jnp.float32),
            pltpu.VMEM((128,), jnp.int32),
        ],
    )
    return run(xflat, cv, scv, zk)


def kernel(x, coefficients_vect, scaling_coeffs_vect, zero_knot_indexes):
    out = _spline_sc(
        x.reshape(-1),
        coefficients_vect,
        scaling_coeffs_vect.reshape(-1),
        zero_knot_indexes.astype(jnp.int32),
    )
    return out.reshape(x.shape)


# speedup vs baseline: 1088.5233x; 1.7733x over previous
import jax
import jax.numpy as jnp
from jax import lax
from jax.experimental import pallas as pl
from jax.experimental.pallas import tpu as pltpu
from jax.experimental.pallas import tpu_sc as plsc

_NUM_ACT = 96
_SIZE = 101
_HALF = _SIZE // 2
_INV_GRID = 12.5

_N = 4 * _NUM_ACT * 224 * 224
_NTILES = 32
_PER_TILE = _N // _NTILES
_SLAB = 224 * 224
_CH = _SLAB // 2
_CHUNKS = _PER_TILE // _CH
_VREGS = _CH // 16


def _tec_body(x_hbm, cv_hbm, sc_hbm, zk_hbm, out_hbm,
              xb0, xb1, ob0, ob1, cv_v, sc_v, zk_v,
              isem0, isem1, osem0, osem1):
    info = plsc.get_sparse_core_info()
    nc = info.num_cores
    wid = lax.axis_index("s") * nc + lax.axis_index("c")
    tile_off = wid * _PER_TILE

    xbufs = (xb0, xb1)
    obufs = (ob0, ob1)
    isems = (isem0, isem1)
    osems = (osem0, osem1)

    def in_copy(k, b):
        return pltpu.make_async_copy(
            x_hbm.at[pl.ds(tile_off + k * _CH, _CH)], xbufs[b], isems[b])

    def out_copy(k, b):
        return pltpu.make_async_copy(
            obufs[b], out_hbm.at[pl.ds(tile_off + k * _CH, _CH)], osems[b])

    pltpu.sync_copy(cv_hbm, cv_v.at[pl.ds(0, _NUM_ACT * _SIZE)])
    pltpu.sync_copy(sc_hbm, sc_v.at[pl.ds(0, _NUM_ACT)])
    pltpu.sync_copy(zk_hbm, zk_v.at[pl.ds(0, _NUM_ACT)])

    in_copy(0, 0).start()
    in_copy(1, 1).start()

    def compute(k, xbuf, obuf):
        c = ((wid * _CHUNKS + k) // (_SLAB // _CH)) % _NUM_ACT
        ci = jnp.full((16,), c, jnp.int32)
        scv = plsc.load_gather(sc_v, [ci])
        zkv = plsc.load_gather(zk_v, [ci])
        a = scv * _INV_GRID
        rs = 1.0 / scv
        base = zkv - _HALF

        @plsc.parallel_loop(0, _VREGS, 1, unroll=8)
        def vec(i):
            xv = xbuf[pl.ds(i * 16, 16)]
            s = xv * a + float(_HALF)
            u = jnp.clip(s, 0.0, float(_SIZE - 2))
            iu = u.astype(jnp.int32)
            fu = iu.astype(jnp.float32)
            fr = s - fu
            ix = base + iu
            g0 = plsc.load_gather(cv_v, [ix])
            g1 = plsc.load_gather(cv_v, [ix + 1])
            obuf[pl.ds(i * 16, 16)] = (g1 * fr + g0 * (1.0 - fr)) * rs

    def pair(p, carry):
        k0 = p * 2
        for b in range(2):
            k = k0 + b
            in_copy(k, b).wait()

            @pl.when(k >= 2)
            def _wait_out():
                out_copy(k - 2, b).wait()

            compute(k, xbufs[b], obufs[b])
            out_copy(k, b).start()

            @pl.when(k + 2 < _CHUNKS)
            def _next_in():
                in_copy(k + 2, b).start()
        return carry

    lax.fori_loop(0, _CHUNKS // 2, pair, 0)
    out_copy(_CHUNKS - 2, 0).wait()
    out_copy(_CHUNKS - 1, 1).wait()


@jax.jit
def _spline_sc(xflat, cv, scv, zk):
    run = pl.kernel(
        _tec_body,
        out_type=jax.ShapeDtypeStruct((_N,), jnp.float32),
        mesh=plsc.VectorSubcoreMesh(core_axis_name="c", subcore_axis_name="s"),
        compiler_params=pltpu.CompilerParams(needs_layout_passes=False),
        scratch_types=[
            pltpu.VMEM((_CH,), jnp.float32),
            pltpu.VMEM((_CH,), jnp.float32),
            pltpu.VMEM((_CH,), jnp.float32),
            pltpu.VMEM((_CH,), jnp.float32),
            pltpu.VMEM((9728,), jnp.float32),
            pltpu.VMEM((128,), jnp.float32),
            pltpu.VMEM((128,), jnp.int32),
            pltpu.SemaphoreType.DMA,
            pltpu.SemaphoreType.DMA,
            pltpu.SemaphoreType.DMA,
            pltpu.SemaphoreType.DMA,
        ],
    )
    return run(xflat, cv, scv, zk)


def kernel(x, coefficients_vect, scaling_coeffs_vect, zero_knot_indexes):
    out = _spline_sc(
        x.reshape(-1),
        coefficients_vect,
        scaling_coeffs_vect.reshape(-1),
        zero_knot_indexes.astype(jnp.int32),
    )
    return out.reshape(x.shape)


# speedup vs baseline: 1174.6875x; 1.0792x over previous
import jax
import jax.numpy as jnp
from jax import lax
from jax.experimental import pallas as pl
from jax.experimental.pallas import tpu as pltpu
from jax.experimental.pallas import tpu_sc as plsc

_NUM_ACT = 96
_SIZE = 101
_HALF = _SIZE // 2
_INV_GRID = 12.5
_CWIN = 128

_N = 4 * _NUM_ACT * 224 * 224
_NTILES = 32
_PER_TILE = _N // _NTILES
_SLAB = 224 * 224
_CH = _SLAB // 4
_CPS = _SLAB // _CH
_CHUNKS = _PER_TILE // _CH
_VREGS = _CH // 16


def _tec_body(x_hbm, cv_hbm, sc_hbm, zk_hbm, out_hbm,
              xb0, xb1, ob0, ob1, cv_raw, sc_v, zk_v, cvp, dp,
              isem0, isem1, osem0, osem1):
    info = plsc.get_sparse_core_info()
    nc = info.num_cores
    wid = lax.axis_index("s") * nc + lax.axis_index("c")
    tile_off = wid * _PER_TILE

    xbufs = (xb0, xb1)
    obufs = (ob0, ob1)
    isems = (isem0, isem1)
    osems = (osem0, osem1)

    def in_copy(k, b):
        return pltpu.make_async_copy(
            x_hbm.at[pl.ds(tile_off + k * _CH, _CH)], xbufs[b], isems[b])

    def out_copy(k, b):
        return pltpu.make_async_copy(
            obufs[b], out_hbm.at[pl.ds(tile_off + k * _CH, _CH)], osems[b])

    in_copy(0, 0).start()
    in_copy(1, 1).start()

    pltpu.sync_copy(cv_hbm, cv_raw.at[pl.ds(0, _NUM_ACT * _SIZE)])
    pltpu.sync_copy(sc_hbm, sc_v.at[pl.ds(0, _NUM_ACT)])
    pltpu.sync_copy(zk_hbm, zk_v.at[pl.ds(0, _NUM_ACT)])

    lane = lax.iota(jnp.int32, 16)

    def prep(c, carry):
        ci = jnp.full((16,), c, jnp.int32)
        zkv = plsc.load_gather(zk_v, [ci])
        rsv = 1.0 / plsc.load_gather(sc_v, [ci])
        b0 = zkv - _HALF
        for j in range(_CWIN // 16):
            src = b0 + (j * 16) + lane
            v0 = plsc.load_gather(cv_raw, [src])
            v1 = plsc.load_gather(cv_raw, [src + 1])
            dst = c * _CWIN + j * 16
            cvp[pl.ds(dst, 16)] = v0 * rsv
            dp[pl.ds(dst, 16)] = (v1 - v0) * rsv
        return carry

    lax.fori_loop(0, _NUM_ACT, prep, 0)

    def compute(k, xbuf, obuf):
        c = ((wid * _CHUNKS + k) // _CPS) % _NUM_ACT
        ci = jnp.full((16,), c, jnp.int32)
        scv = plsc.load_gather(sc_v, [ci])
        a = scv * _INV_GRID
        cb = jnp.full((16,), c * _CWIN, jnp.int32)

        @plsc.parallel_loop(0, _VREGS, 1, unroll=8)
        def vec(i):
            xv = xbuf[pl.ds(i * 16, 16)]
            s = xv * a + float(_HALF)
            u = jnp.clip(s, 0.0, float(_SIZE - 2))
            iu = u.astype(jnp.int32)
            fu = iu.astype(jnp.float32)
            fr = s - fu
            ix = cb + iu
            g0 = plsc.load_gather(cvp, [ix])
            dl = plsc.load_gather(dp, [ix])
            obuf[pl.ds(i * 16, 16)] = g0 + fr * dl

    def pair(p, carry):
        k0 = p * 2
        for b in range(2):
            k = k0 + b
            in_copy(k, b).wait()

            @pl.when(k >= 2)
            def _wait_out():
                out_copy(k - 2, b).wait()

            compute(k, xbufs[b], obufs[b])
            out_copy(k, b).start()

            @pl.when(k + 2 < _CHUNKS)
            def _next_in():
                in_copy(k + 2, b).start()
        return carry

    lax.fori_loop(0, _CHUNKS // 2, pair, 0)
    out_copy(_CHUNKS - 2, 0).wait()
    out_copy(_CHUNKS - 1, 1).wait()


@jax.jit
def _spline_sc(xflat, cv, scv, zk):
    run = pl.kernel(
        _tec_body,
        out_type=jax.ShapeDtypeStruct((_N,), jnp.float32),
        mesh=plsc.VectorSubcoreMesh(core_axis_name="c", subcore_axis_name="s"),
        compiler_params=pltpu.CompilerParams(needs_layout_passes=False),
        scratch_types=[
            pltpu.VMEM((_CH,), jnp.float32),
            pltpu.VMEM((_CH,), jnp.float32),
            pltpu.VMEM((_CH,), jnp.float32),
            pltpu.VMEM((_CH,), jnp.float32),
            pltpu.VMEM((9728,), jnp.float32),
            pltpu.VMEM((128,), jnp.float32),
            pltpu.VMEM((128,), jnp.int32),
            pltpu.VMEM((_NUM_ACT * _CWIN,), jnp.float32),
            pltpu.VMEM((_NUM_ACT * _CWIN,), jnp.float32),
            pltpu.SemaphoreType.DMA,
            pltpu.SemaphoreType.DMA,
            pltpu.SemaphoreType.DMA,
            pltpu.SemaphoreType.DMA,
        ],
    )
    return run(xflat, cv, scv, zk)


def kernel(x, coefficients_vect, scaling_coeffs_vect, zero_knot_indexes):
    out = _spline_sc(
        x.reshape(-1),
        coefficients_vect,
        scaling_coeffs_vect.reshape(-1),
        zero_knot_indexes.astype(jnp.int32),
    )
    return out.reshape(x.shape)


# speedup vs baseline: 1194.1585x; 1.0166x over previous
import jax
import jax.numpy as jnp
from jax import lax
from jax.experimental import pallas as pl
from jax.experimental.pallas import tpu as pltpu
from jax.experimental.pallas import tpu_sc as plsc

_NUM_ACT = 96
_SIZE = 101
_HALF = _SIZE // 2
_INV_GRID = 12.5
_CWIN = 128

_N = 4 * _NUM_ACT * 224 * 224
_NTILES = 32
_PER_TILE = _N // _NTILES
_SLAB = 224 * 224
_CH = _SLAB // 8
_CPS = _SLAB // _CH
_CHUNKS = _PER_TILE // _CH
_VREGS = _CH // 16
_NBUF = 4


def _tec_body(x_hbm, cv_hbm, sc_hbm, zk_hbm, out_hbm,
              xb0, xb1, xb2, xb3, ob0, ob1, ob2, ob3,
              cv_raw, sc_v, zk_v, cvp, dp,
              is0, is1, is2, is3, os0, os1, os2, os3):
    info = plsc.get_sparse_core_info()
    nc = info.num_cores
    wid = lax.axis_index("s") * nc + lax.axis_index("c")

    xbufs = (xb0, xb1, xb2, xb3)
    obufs = (ob0, ob1, ob2, ob3)
    isems = (is0, is1, is2, is3)
    osems = (os0, os1, os2, os3)

    def in_copy(k, b):
        return pltpu.make_async_copy(
            x_hbm.at[wid * _CHUNKS + k], xbufs[b], isems[b])

    def out_copy(k, b):
        return pltpu.make_async_copy(
            obufs[b], out_hbm.at[wid * _CHUNKS + k], osems[b])

    for b in range(_NBUF):
        in_copy(b, b).start()

    pltpu.sync_copy(cv_hbm, cv_raw.at[pl.ds(0, _NUM_ACT * _SIZE)])
    pltpu.sync_copy(sc_hbm, sc_v.at[pl.ds(0, _NUM_ACT)])
    pltpu.sync_copy(zk_hbm, zk_v.at[pl.ds(0, _NUM_ACT)])

    lane = lax.iota(jnp.int32, 16)

    def prep(c, carry):
        ci = jnp.full((16,), c, jnp.int32)
        zkv = plsc.load_gather(zk_v, [ci])
        rsv = 1.0 / plsc.load_gather(sc_v, [ci])
        b0 = zkv - _HALF
        for j in range(_CWIN // 16):
            src = b0 + (j * 16) + lane
            v0 = plsc.load_gather(cv_raw, [src])
            v1 = plsc.load_gather(cv_raw, [src + 1])
            dst = c * _CWIN + j * 16
            cvp[pl.ds(dst, 16)] = v0 * rsv
            dp[pl.ds(dst, 16)] = (v1 - v0) * rsv
        return carry

    lax.fori_loop(0, _NUM_ACT, prep, 0)

    def compute(k, xbuf, obuf):
        c = ((wid * _CHUNKS + k) // _CPS) % _NUM_ACT
        ci = jnp.full((16,), c, jnp.int32)
        scv = plsc.load_gather(sc_v, [ci])
        a = scv * _INV_GRID
        cb = jnp.full((16,), c * _CWIN, jnp.int32)

        @plsc.parallel_loop(0, _VREGS, 1, unroll=8)
        def vec(i):
            xv = xbuf[pl.ds(i * 16, 16)]
            s = xv * a + float(_HALF)
            u = jnp.clip(s, 0.0, float(_SIZE - 2))
            iu = u.astype(jnp.int32)
            fu = iu.astype(jnp.float32)
            fr = s - fu
            ix = cb + iu
            g0 = plsc.load_gather(cvp, [ix])
            dl = plsc.load_gather(dp, [ix])
            obuf[pl.ds(i * 16, 16)] = g0 + fr * dl

    def ring(p, carry):
        k0 = p * _NBUF
        for b in range(_NBUF):
            k = k0 + b
            in_copy(k, b).wait()

            @pl.when(k >= _NBUF)
            def _wait_out():
                out_copy(k - _NBUF, b).wait()

            compute(k, xbufs[b], obufs[b])
            out_copy(k, b).start()

            @pl.when(k + _NBUF < _CHUNKS)
            def _next_in():
                in_copy(k + _NBUF, b).start()
        return carry

    lax.fori_loop(0, _CHUNKS // _NBUF, ring, 0)
    for b in range(_NBUF):
        out_copy(_CHUNKS - _NBUF + b, b).wait()


@jax.jit
def _spline_sc(xflat, cv, scv, zk):
    run = pl.kernel(
        _tec_body,
        out_type=jax.ShapeDtypeStruct((_N // _CH, _CH), jnp.float32),
        mesh=plsc.VectorSubcoreMesh(core_axis_name="c", subcore_axis_name="s"),
        compiler_params=pltpu.CompilerParams(needs_layout_passes=False),
        scratch_types=(
            [pltpu.VMEM((_CH,), jnp.float32)] * (2 * _NBUF)
            + [
                pltpu.VMEM((9728,), jnp.float32),
                pltpu.VMEM((128,), jnp.float32),
                pltpu.VMEM((128,), jnp.int32),
                pltpu.VMEM((_NUM_ACT * _CWIN,), jnp.float32),
                pltpu.VMEM((_NUM_ACT * _CWIN,), jnp.float32),
            ]
            + [pltpu.SemaphoreType.DMA] * (2 * _NBUF)
        ),
    )
    return run(xflat, cv, scv, zk)


def kernel(x, coefficients_vect, scaling_coeffs_vect, zero_knot_indexes):
    out = _spline_sc(
        x.reshape(_N // _CH, _CH),
        coefficients_vect,
        scaling_coeffs_vect.reshape(-1),
        zero_knot_indexes.astype(jnp.int32),
    )
    return out.reshape(x.shape)
